# baseline (speedup 1.0000x reference)
import jax
import jax.numpy as jnp
from jax import lax
from jax.experimental import pallas as pl
from jax.experimental.pallas import tpu as pltpu
from jax.experimental.pallas import tpu_sc as plsc

N = 10000
E = 320000
NFEAT = 128
NHID = 64
NCLASS = 16

NC = 2
NS = 16
NW = NC * NS
EPW = E // NW
CHUNK = 80
NCHUNK = EPW // CHUNK
NGRP = CHUNK // 16
NP = 10240
SPR = NP // NS
ZR = 32
BLK = 2000
WR = 128
NEG = -3.0e38

_MESH = plsc.VectorSubcoreMesh(
    core_axis_name="c", subcore_axis_name="s", num_cores=NC, num_subcores=NS)

_SC_PARAMS = pltpu.CompilerParams(needs_layout_passes=False)

_DNUMS = lax.GatherDimensionNumbers(
    offset_dims=(), collapsed_slice_dims=(0,), start_index_map=(0,))


def _dg(v, idx):
    return lax.gather(v, idx[:, None], _DNUMS, (1,),
                      mode=lax.GatherScatterMode.PROMISE_IN_BOUNDS)


def _p0_body(x_ref, lat_ref, w1t_ref, glg_ref, scal_ref,
             g1_ref, r1_ref, f1_ref):
    feat = jnp.dot(x_ref[...], w1t_ref[...],
                   preferred_element_type=jnp.float32)
    lat = lat_ref[...]
    c_ff = scal_ref[0, 0]
    pad = jnp.zeros((feat.shape[0], WR - NHID - NCLASS), jnp.float32)
    g1_ref[...] = jnp.concatenate([feat, lat * glg_ref[...], pad], axis=1)
    r1_ref[...] = jnp.concatenate([feat * c_ff, lat, pad], axis=1)
    f1_ref[...] = feat


_P0 = pl.pallas_call(
    _p0_body,
    grid=(N // BLK,),
    in_specs=[
        pl.BlockSpec((BLK, NFEAT), lambda i: (i, 0)),
        pl.BlockSpec((BLK, NCLASS), lambda i: (i, 0)),
        pl.BlockSpec((NFEAT, NHID), lambda i: (0, 0)),
        pl.BlockSpec((1, NCLASS), lambda i: (0, 0)),
        pl.BlockSpec((1, 8), lambda i: (0, 0)),
    ],
    out_specs=[
        pl.BlockSpec((BLK, WR), lambda i: (i, 0)),
        pl.BlockSpec((BLK, WR), lambda i: (i, 0)),
        pl.BlockSpec((BLK, NHID), lambda i: (i, 0)),
    ],
    out_shape=[
        jax.ShapeDtypeStruct((N, WR), jnp.float32),
        jax.ShapeDtypeStruct((N, WR), jnp.float32),
        jax.ShapeDtypeStruct((N, NHID), jnp.float32),
    ],
)


def _make_nst(f):
    def body(feat_ref, lat_ref, al_ref, ar_ref, scal_ref, ns_ref):
        feat = feat_ref[...]
        lat = lat_ref[...]
        c_f2 = scal_ref[0, 1]
        c_l2 = scal_ref[0, 2]
        lr = jnp.where(feat > 0, feat, 0.2 * feat)
        el = jnp.sum(lr * al_ref[...], axis=1)
        er = jnp.sum(lr * ar_ref[...], axis=1)
        f2 = jnp.sum(feat * feat, axis=1)
        l2 = jnp.sum(lat * lat, axis=1)
        avec = el - c_f2 * f2 - c_l2 * l2
        bvec = er - c_f2 * f2 - c_l2 * l2
        ns_ref[...] = jnp.concatenate(
            [avec[None], bvec[None],
             jnp.zeros((6, avec.shape[0]), jnp.float32)], axis=0)

    return pl.pallas_call(
        body,
        in_specs=[
            pl.BlockSpec((N, f), lambda: (0, 0)),
            pl.BlockSpec((N, NCLASS), lambda: (0, 0)),
            pl.BlockSpec((1, f), lambda: (0, 0)),
            pl.BlockSpec((1, f), lambda: (0, 0)),
            pl.BlockSpec((1, 8), lambda: (0, 0)),
        ],
        out_specs=pl.BlockSpec((8, N), lambda: (0, 0)),
        out_shape=jax.ShapeDtypeStruct((8, N), jnp.float32),
    )


_NST1 = _make_nst(NHID)
_NST2 = _make_nst(NCLASS)


def _pmx_body(mp_ref, out_ref):
    m = jnp.max(mp_ref[...], axis=0, keepdims=True)
    out_ref[...] = jnp.broadcast_to(m, out_ref.shape)


_PMX = pl.pallas_call(
    _pmx_body,
    in_specs=[pl.BlockSpec((NW, N), lambda: (0, 0))],
    out_specs=pl.BlockSpec((8, N), lambda: (0, 0)),
    out_shape=jax.ShapeDtypeStruct((8, N), jnp.float32),
)


def _p2_body(ad_ref, f1_ref, lat_ref, w2t_ref, glg_ref,
             scal_ref, g2_ref, r2_ref, f2_ref):
    acc = ad_ref[0] + ad_ref[1]
    feat1 = f1_ref[...]
    den = acc[:, NHID]
    indeg = acc[:, NHID + 1]
    c_ff = scal_ref[0, 0]
    c_eps = scal_ref[0, 3]
    s = acc[:, :NHID]
    rst = jnp.where((den > 0)[:, None],
                    s / jnp.where(den > 0, den, 1.0)[:, None], 0.0)
    rst = rst + feat1 * (c_eps / (indeg + 1e-9))[:, None]
    rst = jnp.where(rst > 0, rst, jnp.exp(rst) - 1.0)
    feat2 = jnp.dot(rst, w2t_ref[...], preferred_element_type=jnp.float32)
    lat = lat_ref[...]
    pad = jnp.zeros((feat2.shape[0], WR - 2 * NCLASS), jnp.float32)
    g2_ref[...] = jnp.concatenate([feat2, lat * glg_ref[...], pad], axis=1)
    r2_ref[...] = jnp.concatenate([feat2 * c_ff, lat, pad], axis=1)
    f2_ref[...] = feat2


_P2 = pl.pallas_call(
    _p2_body,
    grid=(N // BLK,),
    in_specs=[
        pl.BlockSpec((2, BLK, WR), lambda i: (0, i, 0)),
        pl.BlockSpec((BLK, NHID), lambda i: (i, 0)),
        pl.BlockSpec((BLK, NCLASS), lambda i: (i, 0)),
        pl.BlockSpec((NHID, NCLASS), lambda i: (0, 0)),
        pl.BlockSpec((1, NCLASS), lambda i: (0, 0)),
        pl.BlockSpec((1, 8), lambda i: (0, 0)),
    ],
    out_specs=[
        pl.BlockSpec((BLK, WR), lambda i: (i, 0)),
        pl.BlockSpec((BLK, WR), lambda i: (i, 0)),
        pl.BlockSpec((BLK, NCLASS), lambda i: (i, 0)),
    ],
    out_shape=[
        jax.ShapeDtypeStruct((N, WR), jnp.float32),
        jax.ShapeDtypeStruct((N, WR), jnp.float32),
        jax.ShapeDtypeStruct((N, NCLASS), jnp.float32),
    ],
)


def _p4_body(ad_ref, f2_ref, scal_ref, out_ref):
    acc = ad_ref[0] + ad_ref[1]
    den = acc[:, NCLASS]
    indeg = acc[:, NCLASS + 1]
    s = acc[:, :NCLASS]
    c_eps = scal_ref[0, 0]
    rst = jnp.where((den > 0)[:, None],
                    s / jnp.where(den > 0, den, 1.0)[:, None], 0.0)
    rst = rst + f2_ref[...] * (c_eps / (indeg + 1e-9))[:, None]
    m = jnp.max(rst, axis=1, keepdims=True)
    lse = jnp.log(jnp.sum(jnp.exp(rst - m), axis=1, keepdims=True)) + m
    out_ref[...] = rst - lse


_P4 = pl.pallas_call(
    _p4_body,
    grid=(N // BLK,),
    in_specs=[
        pl.BlockSpec((2, BLK, WR), lambda i: (0, i, 0)),
        pl.BlockSpec((BLK, NCLASS), lambda i: (i, 0)),
        pl.BlockSpec((1, 8), lambda i: (0, 0)),
    ],
    out_specs=pl.BlockSpec((BLK, NCLASS), lambda i: (i, 0)),
    out_shape=jax.ShapeDtypeStruct((N, NCLASS), jnp.float32),
)


def _make_sc_logits(h):
    def body(esrc, edst, gmat, rmat, nsf, logits, mxpart,
             atab, btab, mxtab, srcall, dstall, logall,
             gs0, gd0, gs1, gd1, sem0, sem1):
        cid = lax.axis_index("c")
        sid = lax.axis_index("s")
        wid = sid * NC + cid
        base_e = wid * EPW
        pltpu.sync_copy(nsf.at[pl.ds(0, N)], atab)
        pltpu.sync_copy(nsf.at[pl.ds(N, N)], btab)
        pltpu.sync_copy(esrc.at[pl.ds(base_e, EPW)], srcall)
        pltpu.sync_copy(edst.at[pl.ds(base_e, EPW)], dstall)

        def zinit(i, c):
            mxtab[pl.ds(i * 16, 16)] = jnp.full((16,), NEG, jnp.float32)
            return c
        lax.fori_loop(0, N // 16, zinit, 0)

        io = lax.iota(jnp.int32, 16)
        bufs = ((gs0, gd0, sem0), (gs1, gd1, sem1))

        def issue(ci, b):
            gs, gd, sem = bufs[b]
            pltpu.async_copy(gmat.at[srcall.at[pl.ds(ci * CHUNK, CHUNK)]],
                             gs, sem)
            pltpu.async_copy(rmat.at[dstall.at[pl.ds(ci * CHUNK, CHUNK)]],
                             gd, sem)

        def waitb(b):
            gs, gd, sem = bufs[b]
            pltpu.make_async_copy(
                gmat.at[srcall.at[pl.ds(0, CHUNK)]], gs, sem).wait()
            pltpu.make_async_copy(
                rmat.at[dstall.at[pl.ds(0, CHUNK)]], gd, sem).wait()

        def compute(ci, b):
            gs, gd, _ = bufs[b]
            ebl = ci * CHUNK
            zero16 = jnp.zeros((16,), jnp.int32)
            for g in range(NGRP):
                rows = g * 16 + io
                s16 = srcall[pl.ds(ebl + g * 16, 16)]
                d16 = dstall[pl.ds(ebl + g * 16, 16)]
                a = plsc.load_gather(atab, [s16]) + plsc.load_gather(btab, [d16])
                flat = rows * WR
                for k in range(h):
                    a = a + (plsc.load_gather(gs, [zero16, flat]) *
                             plsc.load_gather(gd, [zero16, flat]))
                    if k + 1 < h:
                        flat = flat + 1
                logall[pl.ds(ebl + g * 16, 16)] = a
                ks, vs = plsc.sort_key_val(d16, a)
                for st in (1, 2, 4, 8):
                    idxs = jnp.maximum(io - st, 0)
                    vsh = _dg(vs, idxs)
                    ksh = _dg(ks, idxs)
                    ok = (ksh == ks) & (io >= st)
                    vs = jnp.where(ok, jnp.maximum(vs, vsh), vs)
                kn = _dg(ks, jnp.minimum(io + 1, 15))
                is_end = (io == 15) | (kn != ks)
                old = plsc.load_gather(mxtab, [ks], mask=is_end)
                plsc.store_scatter(mxtab, [ks], jnp.maximum(old, vs),
                                   mask=is_end)

        issue(0, 0)
        issue(1, 1)

        def pair(jj, c):
            ci0 = 2 * jj
            waitb(0)
            compute(ci0, 0)
            issue(ci0 + 2, 0)
            ci1 = 2 * jj + 1
            waitb(1)
            compute(ci1, 1)

            @pl.when(ci1 + 2 < NCHUNK)
            def _():
                issue(ci1 + 2, 1)
            return c
        lax.fori_loop(0, NCHUNK // 2, pair, 0)
        waitb(0)
        compute(NCHUNK - 1, 0)

        pltpu.sync_copy(mxtab, mxpart.at[pl.ds(wid * N, N)])
        pltpu.sync_copy(logall, logits.at[pl.ds(base_e, EPW)])

    return pl.kernel(
        body,
        out_type=(jax.ShapeDtypeStruct((E,), jnp.float32),
                  jax.ShapeDtypeStruct((NW * N,), jnp.float32)),
        mesh=_MESH,
        compiler_params=_SC_PARAMS,
        scratch_types=[
            pltpu.VMEM((N,), jnp.float32),
            pltpu.VMEM((N,), jnp.float32),
            pltpu.VMEM((N,), jnp.float32),
            pltpu.VMEM((EPW,), jnp.int32),
            pltpu.VMEM((EPW,), jnp.int32),
            pltpu.VMEM((EPW,), jnp.float32),
            pltpu.VMEM((CHUNK, WR), jnp.float32),
            pltpu.VMEM((CHUNK, WR), jnp.float32),
            pltpu.VMEM((CHUNK, WR), jnp.float32),
            pltpu.VMEM((CHUNK, WR), jnp.float32),
            pltpu.SemaphoreType.DMA,
            pltpu.SemaphoreType.DMA,
        ],
    )


def _make_sc_agg(f):
    def body(esrc, edst, gmat, logits, mxf, accden,
             mxtab, sv0, sv1, dv0, dv1, lv0, lv1, dc0, dc1,
             fs0, fs1, zb, accsp,
             semi0, semi1, semf0, semf1, sems0, sems1):
        cid = lax.axis_index("c")
        sid = lax.axis_index("s")
        base_e = (sid * NC + cid) * EPW
        pltpu.sync_copy(mxf.at[pl.ds(0, N)], mxtab)

        def zrow(i, c):
            for j in range(WR // 16):
                zb[i, pl.ds(j * 16, 16)] = jnp.zeros((16,), jnp.float32)
            return c
        lax.fori_loop(0, ZR, zrow, 0)
        for t in range(SPR // ZR):
            pltpu.sync_copy(zb, accsp.at[pl.ds(sid * SPR + t * ZR, ZR)])
        plsc.subcore_barrier()

        io = lax.iota(jnp.int32, 16)

        ibufs = ((sv0, dv0, lv0, semi0), (sv1, dv1, lv1, semi1))
        fbufs = ((fs0, semf0), (fs1, semf1))
        sbufs = ((fs0, dc0, sems0), (fs1, dc1, sems1))

        def issue_i(ci, b):
            sv, dv, lv, sem = ibufs[b]
            eb = base_e + ci * CHUNK
            pltpu.async_copy(esrc.at[pl.ds(eb, CHUNK)], sv, sem)
            pltpu.async_copy(edst.at[pl.ds(eb, CHUNK)], dv, sem)
            pltpu.async_copy(logits.at[pl.ds(eb, CHUNK)], lv, sem)

        def wait_i(b):
            sv, dv, lv, sem = ibufs[b]
            pltpu.make_async_copy(esrc.at[pl.ds(0, CHUNK)], sv, sem).wait()
            pltpu.make_async_copy(edst.at[pl.ds(0, CHUNK)], dv, sem).wait()
            pltpu.make_async_copy(logits.at[pl.ds(0, CHUNK)], lv, sem).wait()

        def issue_f(b):
            sv = ibufs[b][0]
            fs, sem = fbufs[b]
            pltpu.async_copy(gmat.at[sv], fs, sem)

        def wait_f(b):
            sv = ibufs[b][0]
            fs, sem = fbufs[b]
            pltpu.make_async_copy(gmat.at[sv], fs, sem).wait()

        def issue_s(b):
            fs, dc, sem = sbufs[b]
            pltpu.async_copy(fs, accsp.at[dc], sem, add=True)

        def wait_s(b):
            fs, dc, sem = sbufs[b]
            pltpu.make_async_copy(fs, accsp.at[dc], sem).wait()

        def compute(b):
            fs, _ = fbufs[b]
            dv = ibufs[b][1]
            lv = ibufs[b][2]
            dc = sbufs[b][1]
            ones = jnp.ones((16,), jnp.float32)
            for g in range(NGRP):
                rows = g * 16 + io
                d16 = dv[pl.ds(g * 16, 16)]
                lg = lv[pl.ds(g * 16, 16)]
                dc[pl.ds(g * 16, 16)] = d16
                m = plsc.load_gather(mxtab, [d16])
                ex = jnp.exp(lg - m)
                ll = jnp.zeros((16,), jnp.int32)
                for l in range(16):
                    er = g * 16 + l
                    sp = _dg(ex, ll)
                    for r in range(f // 16):
                        fs[er, pl.ds(r * 16, 16)] = (
                            fs[er, pl.ds(r * 16, 16)] * sp)
                    if l + 1 < 16:
                        ll = ll + 1
                plsc.store_scatter(fs, [rows, jnp.full((16,), f, jnp.int32)],
                                   ex)
                plsc.store_scatter(fs,
                                   [rows, jnp.full((16,), f + 1, jnp.int32)],
                                   ones)
            issue_s(b)

        issue_i(0, 0)
        issue_i(1, 1)
        wait_i(0)
        issue_f(0)
        wait_f(0)
        compute(0)
        issue_i(2, 0)
        wait_i(1)
        issue_f(1)
        wait_f(1)
        compute(1)
        issue_i(3, 1)
        wait_i(0)
        wait_s(0)
        issue_f(0)

        def pair(jj, c):
            wait_f(0)
            compute(0)
            issue_i(2 * jj + 2, 0)
            wait_i(1)
            wait_s(1)
            issue_f(1)
            wait_f(1)
            compute(1)
            issue_i(2 * jj + 3, 1)
            wait_i(0)
            wait_s(0)
            issue_f(0)
            return c
        lax.fori_loop(1, 61, pair, 0)

        wait_f(0)
        compute(0)
        issue_i(124, 0)
        wait_i(1)
        wait_s(1)
        issue_f(1)
        wait_f(1)
        compute(1)
        wait_i(0)
        wait_s(0)
        issue_f(0)
        wait_f(0)
        compute(0)
        wait_s(1)
        wait_s(0)
        plsc.subcore_barrier()
        pltpu.sync_copy(accsp.at[pl.ds(sid * SPR, SPR)],
                        accden.at[cid, pl.ds(sid * SPR, SPR)])

    return pl.kernel(
        body,
        out_type=jax.ShapeDtypeStruct((NC, NP, WR), jnp.float32),
        mesh=_MESH,
        compiler_params=_SC_PARAMS,
        scratch_types=[
            pltpu.VMEM((N,), jnp.float32),
            pltpu.VMEM((CHUNK,), jnp.int32),
            pltpu.VMEM((CHUNK,), jnp.int32),
            pltpu.VMEM((CHUNK,), jnp.int32),
            pltpu.VMEM((CHUNK,), jnp.int32),
            pltpu.VMEM((CHUNK,), jnp.float32),
            pltpu.VMEM((CHUNK,), jnp.float32),
            pltpu.VMEM((CHUNK,), jnp.int32),
            pltpu.VMEM((CHUNK,), jnp.int32),
            pltpu.VMEM((CHUNK, WR), jnp.float32),
            pltpu.VMEM((CHUNK, WR), jnp.float32),
            pltpu.VMEM((ZR, WR), jnp.float32),
            pltpu.VMEM_SHARED((NP, WR), jnp.float32),
            pltpu.SemaphoreType.DMA,
            pltpu.SemaphoreType.DMA,
            pltpu.SemaphoreType.DMA,
            pltpu.SemaphoreType.DMA,
            pltpu.SemaphoreType.DMA,
            pltpu.SemaphoreType.DMA,
        ],
    )


_SCL1 = _make_sc_logits(NHID + NCLASS)
_SCL2 = _make_sc_logits(2 * NCLASS)
_SCA1 = _make_sc_agg(NHID)
_SCA2 = _make_sc_agg(NCLASS)


def _layer_scalars(beta, aw, eps, sa):
    w = jax.nn.softmax(aw, axis=1)
    w0 = w[0, 0]
    w1 = w[0, 1]
    bw = 2.0 / (jnp.exp(-beta[0, 0]) + 1.0)
    c_ff = 2.0 * bw * w0
    c_f2 = bw * w0
    c_l2 = bw * w1
    glg = sa + 2.0 * bw * w1
    c_eps = 1e-9 / (jnp.exp(-eps[0, 0]) + 1.0)
    return c_ff, c_f2, c_l2, glg, c_eps


def _pack8(*vals):
    v = list(vals) + [jnp.float32(0.0)] * (8 - len(vals))
    return jnp.stack([jnp.asarray(x, jnp.float32) for x in v]).reshape(1, 8)


def kernel(x, edge_index, latp, W1, attn_l1, attn_r1, s_attn1, beta1, aw1,
           eps1, W2, attn_l2, attn_r2, s_attn2, beta2, aw2, eps2):
    esrc = edge_index[0].astype(jnp.int32)
    edst = edge_index[1].astype(jnp.int32)
    c_ff1, c_f21, c_l21, glg1, c_eps1 = _layer_scalars(beta1, aw1, eps1,
                                                       s_attn1)
    c_ff2, c_f22, c_l22, glg2, c_eps2 = _layer_scalars(beta2, aw2, eps2,
                                                       s_attn2)
    scal0 = _pack8(c_ff1, c_f21, c_l21)
    scal2 = _pack8(c_ff2, c_f22, c_l22, c_eps1)
    scal4 = _pack8(c_eps2)

    g1, r1, f1 = _P0(x, latp, W1.T, glg1, scal0)
    ns1 = _NST1(f1, latp, attn_l1.reshape(1, NHID),
                attn_r1.reshape(1, NHID), scal0)
    logits1, mxp1 = _SCL1(esrc, edst, g1, r1, ns1.reshape(-1))
    mx1 = _PMX(mxp1.reshape(NW, N))
    ad1 = _SCA1(esrc, edst, g1, logits1, mx1.reshape(-1))
    g2, r2, f2 = _P2(ad1, f1, latp, W2.T, glg2, scal2)
    ns2 = _NST2(f2, latp, attn_l2.reshape(1, NCLASS),
                attn_r2.reshape(1, NCLASS), scal2)
    logits2, mxp2 = _SCL2(esrc, edst, g2, r2, ns2.reshape(-1))
    mx2 = _PMX(mxp2.reshape(NW, N))
    ad2 = _SCA2(esrc, edst, g2, logits2, mx2.reshape(-1))
    out = _P4(ad2, f2, scal4)
    return out


# speedup vs baseline: 1.0084x; 1.0084x over previous
import jax
import jax.numpy as jnp
from jax import lax
from jax.experimental import pallas as pl
from jax.experimental.pallas import tpu as pltpu
from jax.experimental.pallas import tpu_sc as plsc

N = 10000
E = 320000
NFEAT = 128
NHID = 64
NCLASS = 16

NC = 2
NS = 16
NW = NC * NS
EPW = E // NW
CHUNK = 80
NCHUNK = EPW // CHUNK
NGRP = CHUNK // 16
NP = 10240
SPR = NP // NS
ZR = 32
BLK = 2000
WR = 128
NEG = -3.0e38

_MESH = plsc.VectorSubcoreMesh(
    core_axis_name="c", subcore_axis_name="s", num_cores=NC, num_subcores=NS)

_SC_PARAMS = pltpu.CompilerParams(needs_layout_passes=False)

_DNUMS = lax.GatherDimensionNumbers(
    offset_dims=(), collapsed_slice_dims=(0,), start_index_map=(0,))


def _dg(v, idx):
    return lax.gather(v, idx[:, None], _DNUMS, (1,),
                      mode=lax.GatherScatterMode.PROMISE_IN_BOUNDS)


def _p0_body(x_ref, lat_ref, w1t_ref, glg_ref, scal_ref,
             g1_ref, r1_ref, f1_ref):
    feat = jnp.dot(x_ref[...], w1t_ref[...],
                   preferred_element_type=jnp.float32)
    lat = lat_ref[...]
    c_ff = scal_ref[0, 0]
    pad = jnp.zeros((feat.shape[0], WR - NHID - NCLASS), jnp.float32)
    g1_ref[...] = jnp.concatenate([feat, lat * glg_ref[...], pad], axis=1)
    r1_ref[...] = jnp.concatenate([feat * c_ff, lat, pad], axis=1)
    f1_ref[...] = feat


_P0 = pl.pallas_call(
    _p0_body,
    grid=(N // BLK,),
    in_specs=[
        pl.BlockSpec((BLK, NFEAT), lambda i: (i, 0)),
        pl.BlockSpec((BLK, NCLASS), lambda i: (i, 0)),
        pl.BlockSpec((NFEAT, NHID), lambda i: (0, 0)),
        pl.BlockSpec((1, NCLASS), lambda i: (0, 0)),
        pl.BlockSpec((1, 8), lambda i: (0, 0)),
    ],
    out_specs=[
        pl.BlockSpec((BLK, WR), lambda i: (i, 0)),
        pl.BlockSpec((BLK, WR), lambda i: (i, 0)),
        pl.BlockSpec((BLK, NHID), lambda i: (i, 0)),
    ],
    out_shape=[
        jax.ShapeDtypeStruct((N, WR), jnp.float32),
        jax.ShapeDtypeStruct((N, WR), jnp.float32),
        jax.ShapeDtypeStruct((N, NHID), jnp.float32),
    ],
)


def _make_nst(f):
    def body(feat_ref, lat_ref, al_ref, ar_ref, scal_ref, ns_ref):
        feat = feat_ref[...]
        lat = lat_ref[...]
        c_f2 = scal_ref[0, 1]
        c_l2 = scal_ref[0, 2]
        lr = jnp.where(feat > 0, feat, 0.2 * feat)
        el = jnp.sum(lr * al_ref[...], axis=1)
        er = jnp.sum(lr * ar_ref[...], axis=1)
        f2 = jnp.sum(feat * feat, axis=1)
        l2 = jnp.sum(lat * lat, axis=1)
        avec = el - c_f2 * f2 - c_l2 * l2
        bvec = er - c_f2 * f2 - c_l2 * l2
        ns_ref[...] = jnp.concatenate(
            [avec[None], bvec[None],
             jnp.zeros((6, avec.shape[0]), jnp.float32)], axis=0)

    return pl.pallas_call(
        body,
        in_specs=[
            pl.BlockSpec((N, f), lambda: (0, 0)),
            pl.BlockSpec((N, NCLASS), lambda: (0, 0)),
            pl.BlockSpec((1, f), lambda: (0, 0)),
            pl.BlockSpec((1, f), lambda: (0, 0)),
            pl.BlockSpec((1, 8), lambda: (0, 0)),
        ],
        out_specs=pl.BlockSpec((8, N), lambda: (0, 0)),
        out_shape=jax.ShapeDtypeStruct((8, N), jnp.float32),
    )


_NST1 = _make_nst(NHID)
_NST2 = _make_nst(NCLASS)


def _pmx_body(mp_ref, out_ref):
    m = jnp.max(mp_ref[...], axis=0, keepdims=True)
    out_ref[...] = jnp.broadcast_to(m, out_ref.shape)


_PMX = pl.pallas_call(
    _pmx_body,
    in_specs=[pl.BlockSpec((NW, N), lambda: (0, 0))],
    out_specs=pl.BlockSpec((8, N), lambda: (0, 0)),
    out_shape=jax.ShapeDtypeStruct((8, N), jnp.float32),
)


def _p2_body(ad_ref, f1_ref, lat_ref, w2t_ref, glg_ref,
             scal_ref, g2_ref, r2_ref, f2_ref):
    acc = ad_ref[0] + ad_ref[1]
    feat1 = f1_ref[...]
    den = acc[:, NHID]
    indeg = acc[:, NHID + 1]
    c_ff = scal_ref[0, 0]
    c_eps = scal_ref[0, 3]
    s = acc[:, :NHID]
    rst = jnp.where((den > 0)[:, None],
                    s / jnp.where(den > 0, den, 1.0)[:, None], 0.0)
    rst = rst + feat1 * (c_eps / (indeg + 1e-9))[:, None]
    rst = jnp.where(rst > 0, rst, jnp.exp(rst) - 1.0)
    feat2 = jnp.dot(rst, w2t_ref[...], preferred_element_type=jnp.float32)
    lat = lat_ref[...]
    pad = jnp.zeros((feat2.shape[0], WR - 2 * NCLASS), jnp.float32)
    g2_ref[...] = jnp.concatenate([feat2, lat * glg_ref[...], pad], axis=1)
    r2_ref[...] = jnp.concatenate([feat2 * c_ff, lat, pad], axis=1)
    f2_ref[...] = feat2


_P2 = pl.pallas_call(
    _p2_body,
    grid=(N // BLK,),
    in_specs=[
        pl.BlockSpec((2, BLK, WR), lambda i: (0, i, 0)),
        pl.BlockSpec((BLK, NHID), lambda i: (i, 0)),
        pl.BlockSpec((BLK, NCLASS), lambda i: (i, 0)),
        pl.BlockSpec((NHID, NCLASS), lambda i: (0, 0)),
        pl.BlockSpec((1, NCLASS), lambda i: (0, 0)),
        pl.BlockSpec((1, 8), lambda i: (0, 0)),
    ],
    out_specs=[
        pl.BlockSpec((BLK, WR), lambda i: (i, 0)),
        pl.BlockSpec((BLK, WR), lambda i: (i, 0)),
        pl.BlockSpec((BLK, NCLASS), lambda i: (i, 0)),
    ],
    out_shape=[
        jax.ShapeDtypeStruct((N, WR), jnp.float32),
        jax.ShapeDtypeStruct((N, WR), jnp.float32),
        jax.ShapeDtypeStruct((N, NCLASS), jnp.float32),
    ],
)


def _p4_body(ad_ref, f2_ref, scal_ref, out_ref):
    acc = ad_ref[0] + ad_ref[1]
    den = acc[:, NCLASS]
    indeg = acc[:, NCLASS + 1]
    s = acc[:, :NCLASS]
    c_eps = scal_ref[0, 0]
    rst = jnp.where((den > 0)[:, None],
                    s / jnp.where(den > 0, den, 1.0)[:, None], 0.0)
    rst = rst + f2_ref[...] * (c_eps / (indeg + 1e-9))[:, None]
    m = jnp.max(rst, axis=1, keepdims=True)
    lse = jnp.log(jnp.sum(jnp.exp(rst - m), axis=1, keepdims=True)) + m
    out_ref[...] = rst - lse


_P4 = pl.pallas_call(
    _p4_body,
    grid=(N // BLK,),
    in_specs=[
        pl.BlockSpec((2, BLK, WR), lambda i: (0, i, 0)),
        pl.BlockSpec((BLK, NCLASS), lambda i: (i, 0)),
        pl.BlockSpec((1, 8), lambda i: (0, 0)),
    ],
    out_specs=pl.BlockSpec((BLK, NCLASS), lambda i: (i, 0)),
    out_shape=jax.ShapeDtypeStruct((N, NCLASS), jnp.float32),
)


def _make_sc_logits(h):
    def body(esrc, edst, gmat, rmat, nsf, logits, mxpart,
             atab, btab, mxtab, srcall, dstall, logall,
             gs0, gd0, gs1, gd1, sem0, sem1):
        cid = lax.axis_index("c")
        sid = lax.axis_index("s")
        wid = sid * NC + cid
        base_e = wid * EPW
        pltpu.sync_copy(nsf.at[pl.ds(0, N)], atab)
        pltpu.sync_copy(nsf.at[pl.ds(N, N)], btab)
        pltpu.sync_copy(esrc.at[pl.ds(base_e, EPW)], srcall)
        pltpu.sync_copy(edst.at[pl.ds(base_e, EPW)], dstall)

        def zinit(i, c):
            mxtab[pl.ds(i * 16, 16)] = jnp.full((16,), NEG, jnp.float32)
            return c
        lax.fori_loop(0, N // 16, zinit, 0)

        io = lax.iota(jnp.int32, 16)
        bufs = ((gs0, gd0, sem0), (gs1, gd1, sem1))

        def issue(ci, b):
            gs, gd, sem = bufs[b]
            pltpu.async_copy(gmat.at[srcall.at[pl.ds(ci * CHUNK, CHUNK)]],
                             gs, sem)
            pltpu.async_copy(rmat.at[dstall.at[pl.ds(ci * CHUNK, CHUNK)]],
                             gd, sem)

        def waitb(b):
            gs, gd, sem = bufs[b]
            pltpu.make_async_copy(
                gmat.at[srcall.at[pl.ds(0, CHUNK)]], gs, sem).wait()
            pltpu.make_async_copy(
                rmat.at[dstall.at[pl.ds(0, CHUNK)]], gd, sem).wait()

        def compute(ci, b):
            gs, gd, _ = bufs[b]
            ebl = ci * CHUNK
            zero16 = jnp.zeros((16,), jnp.int32)
            for g in range(NGRP):
                rows = g * 16 + io
                s16 = srcall[pl.ds(ebl + g * 16, 16)]
                d16 = dstall[pl.ds(ebl + g * 16, 16)]
                flat = rows * WR
                acc4 = [plsc.load_gather(atab, [s16]),
                        plsc.load_gather(btab, [d16]),
                        jnp.zeros((16,), jnp.float32),
                        jnp.zeros((16,), jnp.float32)]
                for k in range(h):
                    acc4[k % 4] = acc4[k % 4] + (
                        plsc.load_gather(gs, [zero16, flat]) *
                        plsc.load_gather(gd, [zero16, flat]))
                    if k + 1 < h:
                        flat = flat + 1
                a = (acc4[0] + acc4[1]) + (acc4[2] + acc4[3])
                logall[pl.ds(ebl + g * 16, 16)] = a
                ks, vs = plsc.sort_key_val(d16, a)
                for st in (1, 2, 4, 8):
                    idxs = jnp.maximum(io - st, 0)
                    vsh = _dg(vs, idxs)
                    ksh = _dg(ks, idxs)
                    ok = (ksh == ks) & (io >= st)
                    vs = jnp.where(ok, jnp.maximum(vs, vsh), vs)
                kn = _dg(ks, jnp.minimum(io + 1, 15))
                is_end = (io == 15) | (kn != ks)
                old = plsc.load_gather(mxtab, [ks], mask=is_end)
                plsc.store_scatter(mxtab, [ks], jnp.maximum(old, vs),
                                   mask=is_end)

        issue(0, 0)
        issue(1, 1)

        def pair(jj, c):
            ci0 = 2 * jj
            waitb(0)
            compute(ci0, 0)
            issue(ci0 + 2, 0)
            ci1 = 2 * jj + 1
            waitb(1)
            compute(ci1, 1)

            @pl.when(ci1 + 2 < NCHUNK)
            def _():
                issue(ci1 + 2, 1)
            return c
        lax.fori_loop(0, NCHUNK // 2, pair, 0)
        waitb(0)
        compute(NCHUNK - 1, 0)

        pltpu.sync_copy(mxtab, mxpart.at[pl.ds(wid * N, N)])
        pltpu.sync_copy(logall, logits.at[pl.ds(base_e, EPW)])

    return pl.kernel(
        body,
        out_type=(jax.ShapeDtypeStruct((E,), jnp.float32),
                  jax.ShapeDtypeStruct((NW * N,), jnp.float32)),
        mesh=_MESH,
        compiler_params=_SC_PARAMS,
        scratch_types=[
            pltpu.VMEM((N,), jnp.float32),
            pltpu.VMEM((N,), jnp.float32),
            pltpu.VMEM((N,), jnp.float32),
            pltpu.VMEM((EPW,), jnp.int32),
            pltpu.VMEM((EPW,), jnp.int32),
            pltpu.VMEM((EPW,), jnp.float32),
            pltpu.VMEM((CHUNK, WR), jnp.float32),
            pltpu.VMEM((CHUNK, WR), jnp.float32),
            pltpu.VMEM((CHUNK, WR), jnp.float32),
            pltpu.VMEM((CHUNK, WR), jnp.float32),
            pltpu.SemaphoreType.DMA,
            pltpu.SemaphoreType.DMA,
        ],
    )


def _make_sc_agg(f):
    def body(esrc, edst, gmat, logits, mxf, accden,
             mxtab, sv0, sv1, dv0, dv1, lv0, lv1, dc0, dc1,
             fs0, fs1, zb, accsp,
             semi0, semi1, semf0, semf1, sems0, sems1):
        cid = lax.axis_index("c")
        sid = lax.axis_index("s")
        base_e = (sid * NC + cid) * EPW
        pltpu.sync_copy(mxf.at[pl.ds(0, N)], mxtab)

        def zrow(i, c):
            for j in range(WR // 16):
                zb[i, pl.ds(j * 16, 16)] = jnp.zeros((16,), jnp.float32)
            return c
        lax.fori_loop(0, ZR, zrow, 0)
        for t in range(SPR // ZR):
            pltpu.sync_copy(zb, accsp.at[pl.ds(sid * SPR + t * ZR, ZR)])
        plsc.subcore_barrier()

        io = lax.iota(jnp.int32, 16)

        ibufs = ((sv0, dv0, lv0, semi0), (sv1, dv1, lv1, semi1))
        fbufs = ((fs0, semf0), (fs1, semf1))
        sbufs = ((fs0, dc0, sems0), (fs1, dc1, sems1))

        def issue_i(ci, b):
            sv, dv, lv, sem = ibufs[b]
            eb = base_e + ci * CHUNK
            pltpu.async_copy(esrc.at[pl.ds(eb, CHUNK)], sv, sem)
            pltpu.async_copy(edst.at[pl.ds(eb, CHUNK)], dv, sem)
            pltpu.async_copy(logits.at[pl.ds(eb, CHUNK)], lv, sem)

        def wait_i(b):
            sv, dv, lv, sem = ibufs[b]
            pltpu.make_async_copy(esrc.at[pl.ds(0, CHUNK)], sv, sem).wait()
            pltpu.make_async_copy(edst.at[pl.ds(0, CHUNK)], dv, sem).wait()
            pltpu.make_async_copy(logits.at[pl.ds(0, CHUNK)], lv, sem).wait()

        def issue_f(b):
            sv = ibufs[b][0]
            fs, sem = fbufs[b]
            pltpu.async_copy(gmat.at[sv], fs, sem)

        def wait_f(b):
            sv = ibufs[b][0]
            fs, sem = fbufs[b]
            pltpu.make_async_copy(gmat.at[sv], fs, sem).wait()

        def issue_s(b):
            fs, dc, sem = sbufs[b]
            pltpu.async_copy(fs, accsp.at[dc], sem, add=True)

        def wait_s(b):
            fs, dc, sem = sbufs[b]
            pltpu.make_async_copy(fs, accsp.at[dc], sem).wait()

        def compute(b):
            fs, _ = fbufs[b]
            dv = ibufs[b][1]
            lv = ibufs[b][2]
            dc = sbufs[b][1]
            ones = jnp.ones((16,), jnp.float32)
            for g in range(NGRP):
                rows = g * 16 + io
                d16 = dv[pl.ds(g * 16, 16)]
                lg = lv[pl.ds(g * 16, 16)]
                dc[pl.ds(g * 16, 16)] = d16
                m = plsc.load_gather(mxtab, [d16])
                ex = jnp.exp(lg - m)
                ll = jnp.zeros((16,), jnp.int32)
                for l in range(16):
                    er = g * 16 + l
                    sp = _dg(ex, ll)
                    for r in range(f // 16):
                        fs[er, pl.ds(r * 16, 16)] = (
                            fs[er, pl.ds(r * 16, 16)] * sp)
                    if l + 1 < 16:
                        ll = ll + 1
                plsc.store_scatter(fs, [rows, jnp.full((16,), f, jnp.int32)],
                                   ex)
                plsc.store_scatter(fs,
                                   [rows, jnp.full((16,), f + 1, jnp.int32)],
                                   ones)
            issue_s(b)

        issue_i(0, 0)
        issue_i(1, 1)
        wait_i(0)
        issue_f(0)
        wait_f(0)
        compute(0)
        issue_i(2, 0)
        wait_i(1)
        issue_f(1)
        wait_f(1)
        compute(1)
        issue_i(3, 1)
        wait_i(0)
        wait_s(0)
        issue_f(0)

        def pair(jj, c):
            wait_f(0)
            compute(0)
            issue_i(2 * jj + 2, 0)
            wait_i(1)
            wait_s(1)
            issue_f(1)
            wait_f(1)
            compute(1)
            issue_i(2 * jj + 3, 1)
            wait_i(0)
            wait_s(0)
            issue_f(0)
            return c
        lax.fori_loop(1, 61, pair, 0)

        wait_f(0)
        compute(0)
        issue_i(124, 0)
        wait_i(1)
        wait_s(1)
        issue_f(1)
        wait_f(1)
        compute(1)
        wait_i(0)
        wait_s(0)
        issue_f(0)
        wait_f(0)
        compute(0)
        wait_s(1)
        wait_s(0)
        plsc.subcore_barrier()
        pltpu.sync_copy(accsp.at[pl.ds(sid * SPR, SPR)],
                        accden.at[cid, pl.ds(sid * SPR, SPR)])

    return pl.kernel(
        body,
        out_type=jax.ShapeDtypeStruct((NC, NP, WR), jnp.float32),
        mesh=_MESH,
        compiler_params=_SC_PARAMS,
        scratch_types=[
            pltpu.VMEM((N,), jnp.float32),
            pltpu.VMEM((CHUNK,), jnp.int32),
            pltpu.VMEM((CHUNK,), jnp.int32),
            pltpu.VMEM((CHUNK,), jnp.int32),
            pltpu.VMEM((CHUNK,), jnp.int32),
            pltpu.VMEM((CHUNK,), jnp.float32),
            pltpu.VMEM((CHUNK,), jnp.float32),
            pltpu.VMEM((CHUNK,), jnp.int32),
            pltpu.VMEM((CHUNK,), jnp.int32),
            pltpu.VMEM((CHUNK, WR), jnp.float32),
            pltpu.VMEM((CHUNK, WR), jnp.float32),
            pltpu.VMEM((ZR, WR), jnp.float32),
            pltpu.VMEM_SHARED((NP, WR), jnp.float32),
            pltpu.SemaphoreType.DMA,
            pltpu.SemaphoreType.DMA,
            pltpu.SemaphoreType.DMA,
            pltpu.SemaphoreType.DMA,
            pltpu.SemaphoreType.DMA,
            pltpu.SemaphoreType.DMA,
        ],
    )


_SCL1 = _make_sc_logits(NHID + NCLASS)
_SCL2 = _make_sc_logits(2 * NCLASS)
_SCA1 = _make_sc_agg(NHID)
_SCA2 = _make_sc_agg(NCLASS)


def _layer_scalars(beta, aw, eps, sa):
    w = jax.nn.softmax(aw, axis=1)
    w0 = w[0, 0]
    w1 = w[0, 1]
    bw = 2.0 / (jnp.exp(-beta[0, 0]) + 1.0)
    c_ff = 2.0 * bw * w0
    c_f2 = bw * w0
    c_l2 = bw * w1
    glg = sa + 2.0 * bw * w1
    c_eps = 1e-9 / (jnp.exp(-eps[0, 0]) + 1.0)
    return c_ff, c_f2, c_l2, glg, c_eps


def _pack8(*vals):
    v = list(vals) + [jnp.float32(0.0)] * (8 - len(vals))
    return jnp.stack([jnp.asarray(x, jnp.float32) for x in v]).reshape(1, 8)


def kernel(x, edge_index, latp, W1, attn_l1, attn_r1, s_attn1, beta1, aw1,
           eps1, W2, attn_l2, attn_r2, s_attn2, beta2, aw2, eps2):
    esrc = edge_index[0].astype(jnp.int32)
    edst = edge_index[1].astype(jnp.int32)
    c_ff1, c_f21, c_l21, glg1, c_eps1 = _layer_scalars(beta1, aw1, eps1,
                                                       s_attn1)
    c_ff2, c_f22, c_l22, glg2, c_eps2 = _layer_scalars(beta2, aw2, eps2,
                                                       s_attn2)
    scal0 = _pack8(c_ff1, c_f21, c_l21)
    scal2 = _pack8(c_ff2, c_f22, c_l22, c_eps1)
    scal4 = _pack8(c_eps2)

    g1, r1, f1 = _P0(x, latp, W1.T, glg1, scal0)
    ns1 = _NST1(f1, latp, attn_l1.reshape(1, NHID),
                attn_r1.reshape(1, NHID), scal0)
    logits1, mxp1 = _SCL1(esrc, edst, g1, r1, ns1.reshape(-1))
    mx1 = _PMX(mxp1.reshape(NW, N))
    ad1 = _SCA1(esrc, edst, g1, logits1, mx1.reshape(-1))
    g2, r2, f2 = _P2(ad1, f1, latp, W2.T, glg2, scal2)
    ns2 = _NST2(f2, latp, attn_l2.reshape(1, NCLASS),
                attn_r2.reshape(1, NCLASS), scal2)
    logits2, mxp2 = _SCL2(esrc, edst, g2, r2, ns2.reshape(-1))
    mx2 = _PMX(mxp2.reshape(NW, N))
    ad2 = _SCA2(esrc, edst, g2, logits2, mx2.reshape(-1))
    out = _P4(ad2, f2, scal4)
    return out


# speedup vs baseline: 1.8813x; 1.8656x over previous
import jax
import jax.numpy as jnp
from jax import lax
from jax.experimental import pallas as pl
from jax.experimental.pallas import tpu as pltpu
from jax.experimental.pallas import tpu_sc as plsc

N = 10000
E = 320000
NFEAT = 128
NHID = 64
NCLASS = 16

NC = 2
NS = 16
NW = NC * NS
EPW = E // NW
CHUNK = 80
NCHUNK = EPW // CHUNK
NGRP = CHUNK // 16
NP = 10240
SPR = NP // NS
ZR = 32
BLK = 2000
WR = 128
NEG = -3.0e38

_MESH = plsc.VectorSubcoreMesh(
    core_axis_name="c", subcore_axis_name="s", num_cores=NC, num_subcores=NS)

_SC_PARAMS = pltpu.CompilerParams(needs_layout_passes=False)

_DNUMS = lax.GatherDimensionNumbers(
    offset_dims=(), collapsed_slice_dims=(0,), start_index_map=(0,))


def _dg(v, idx):
    return lax.gather(v, idx[:, None], _DNUMS, (1,),
                      mode=lax.GatherScatterMode.PROMISE_IN_BOUNDS)


def _p0_body(x_ref, lat_ref, w1t_ref, glg_ref, scal_ref,
             g1_ref, r1_ref, f1_ref):
    feat = jnp.dot(x_ref[...], w1t_ref[...],
                   preferred_element_type=jnp.float32)
    lat = lat_ref[...]
    c_ff = scal_ref[0, 0]
    pad = jnp.zeros((feat.shape[0], WR - NHID - NCLASS), jnp.float32)
    g1_ref[...] = jnp.concatenate([feat, lat * glg_ref[...], pad], axis=1)
    r1_ref[...] = jnp.concatenate([feat * c_ff, lat, pad], axis=1)
    f1_ref[...] = feat


_P0 = pl.pallas_call(
    _p0_body,
    grid=(N // BLK,),
    in_specs=[
        pl.BlockSpec((BLK, NFEAT), lambda i: (i, 0)),
        pl.BlockSpec((BLK, NCLASS), lambda i: (i, 0)),
        pl.BlockSpec((NFEAT, NHID), lambda i: (0, 0)),
        pl.BlockSpec((1, NCLASS), lambda i: (0, 0)),
        pl.BlockSpec((1, 8), lambda i: (0, 0)),
    ],
    out_specs=[
        pl.BlockSpec((BLK, WR), lambda i: (i, 0)),
        pl.BlockSpec((BLK, WR), lambda i: (i, 0)),
        pl.BlockSpec((BLK, NHID), lambda i: (i, 0)),
    ],
    out_shape=[
        jax.ShapeDtypeStruct((N, WR), jnp.float32),
        jax.ShapeDtypeStruct((N, WR), jnp.float32),
        jax.ShapeDtypeStruct((N, NHID), jnp.float32),
    ],
)


def _make_nst(f):
    def body(feat_ref, lat_ref, al_ref, ar_ref, scal_ref, ns_ref):
        feat = feat_ref[...]
        lat = lat_ref[...]
        c_f2 = scal_ref[0, 1]
        c_l2 = scal_ref[0, 2]
        lr = jnp.where(feat > 0, feat, 0.2 * feat)
        el = jnp.sum(lr * al_ref[...], axis=1)
        er = jnp.sum(lr * ar_ref[...], axis=1)
        f2 = jnp.sum(feat * feat, axis=1)
        l2 = jnp.sum(lat * lat, axis=1)
        avec = el - c_f2 * f2 - c_l2 * l2
        bvec = er - c_f2 * f2 - c_l2 * l2
        ns_ref[...] = jnp.concatenate(
            [avec[None], bvec[None],
             jnp.zeros((6, avec.shape[0]), jnp.float32)], axis=0)

    return pl.pallas_call(
        body,
        in_specs=[
            pl.BlockSpec((N, f), lambda: (0, 0)),
            pl.BlockSpec((N, NCLASS), lambda: (0, 0)),
            pl.BlockSpec((1, f), lambda: (0, 0)),
            pl.BlockSpec((1, f), lambda: (0, 0)),
            pl.BlockSpec((1, 8), lambda: (0, 0)),
        ],
        out_specs=pl.BlockSpec((8, N), lambda: (0, 0)),
        out_shape=jax.ShapeDtypeStruct((8, N), jnp.float32),
    )


_NST1 = _make_nst(NHID)
_NST2 = _make_nst(NCLASS)


def _pmx_body(mp_ref, out_ref):
    m = jnp.max(mp_ref[...], axis=0, keepdims=True)
    out_ref[...] = jnp.broadcast_to(m, out_ref.shape)


_PMX = pl.pallas_call(
    _pmx_body,
    in_specs=[pl.BlockSpec((NW, N), lambda: (0, 0))],
    out_specs=pl.BlockSpec((8, N), lambda: (0, 0)),
    out_shape=jax.ShapeDtypeStruct((8, N), jnp.float32),
)


def _p2_body(ad_ref, f1_ref, lat_ref, w2t_ref, glg_ref,
             scal_ref, g2_ref, r2_ref, f2_ref):
    acc = ad_ref[0] + ad_ref[1]
    feat1 = f1_ref[...]
    den = acc[:, NHID]
    indeg = acc[:, NHID + 1]
    c_ff = scal_ref[0, 0]
    c_eps = scal_ref[0, 3]
    s = acc[:, :NHID]
    rst = jnp.where((den > 0)[:, None],
                    s / jnp.where(den > 0, den, 1.0)[:, None], 0.0)
    rst = rst + feat1 * (c_eps / (indeg + 1e-9))[:, None]
    rst = jnp.where(rst > 0, rst, jnp.exp(rst) - 1.0)
    feat2 = jnp.dot(rst, w2t_ref[...], preferred_element_type=jnp.float32)
    lat = lat_ref[...]
    pad = jnp.zeros((feat2.shape[0], WR - 2 * NCLASS), jnp.float32)
    g2_ref[...] = jnp.concatenate([feat2, lat * glg_ref[...], pad], axis=1)
    r2_ref[...] = jnp.concatenate([feat2 * c_ff, lat, pad], axis=1)
    f2_ref[...] = feat2


_P2 = pl.pallas_call(
    _p2_body,
    grid=(N // BLK,),
    in_specs=[
        pl.BlockSpec((2, BLK, WR), lambda i: (0, i, 0)),
        pl.BlockSpec((BLK, NHID), lambda i: (i, 0)),
        pl.BlockSpec((BLK, NCLASS), lambda i: (i, 0)),
        pl.BlockSpec((NHID, NCLASS), lambda i: (0, 0)),
        pl.BlockSpec((1, NCLASS), lambda i: (0, 0)),
        pl.BlockSpec((1, 8), lambda i: (0, 0)),
    ],
    out_specs=[
        pl.BlockSpec((BLK, WR), lambda i: (i, 0)),
        pl.BlockSpec((BLK, WR), lambda i: (i, 0)),
        pl.BlockSpec((BLK, NCLASS), lambda i: (i, 0)),
    ],
    out_shape=[
        jax.ShapeDtypeStruct((N, WR), jnp.float32),
        jax.ShapeDtypeStruct((N, WR), jnp.float32),
        jax.ShapeDtypeStruct((N, NCLASS), jnp.float32),
    ],
)


def _p4_body(ad_ref, f2_ref, scal_ref, out_ref):
    acc = ad_ref[0] + ad_ref[1]
    den = acc[:, NCLASS]
    indeg = acc[:, NCLASS + 1]
    s = acc[:, :NCLASS]
    c_eps = scal_ref[0, 0]
    rst = jnp.where((den > 0)[:, None],
                    s / jnp.where(den > 0, den, 1.0)[:, None], 0.0)
    rst = rst + f2_ref[...] * (c_eps / (indeg + 1e-9))[:, None]
    m = jnp.max(rst, axis=1, keepdims=True)
    lse = jnp.log(jnp.sum(jnp.exp(rst - m), axis=1, keepdims=True)) + m
    out_ref[...] = rst - lse


_P4 = pl.pallas_call(
    _p4_body,
    grid=(N // BLK,),
    in_specs=[
        pl.BlockSpec((2, BLK, WR), lambda i: (0, i, 0)),
        pl.BlockSpec((BLK, NCLASS), lambda i: (i, 0)),
        pl.BlockSpec((1, 8), lambda i: (0, 0)),
    ],
    out_specs=pl.BlockSpec((BLK, NCLASS), lambda i: (i, 0)),
    out_shape=jax.ShapeDtypeStruct((N, NCLASS), jnp.float32),
)


def _make_sc_logits(h):
    def body(esrc, edst, gmat, rmat, nsf, logits, mxpart,
             atab, btab, mxtab, srcall, dstall, logall,
             gs0, gd0, gs1, gd1, sem0, sem1):
        cid = lax.axis_index("c")
        sid = lax.axis_index("s")
        wid = sid * NC + cid
        base_e = wid * EPW
        pltpu.sync_copy(nsf.at[pl.ds(0, N)], atab)
        pltpu.sync_copy(nsf.at[pl.ds(N, N)], btab)
        pltpu.sync_copy(esrc.at[pl.ds(base_e, EPW)], srcall)
        pltpu.sync_copy(edst.at[pl.ds(base_e, EPW)], dstall)

        def zinit(i, c):
            mxtab[pl.ds(i * 16, 16)] = jnp.full((16,), NEG, jnp.float32)
            return c
        lax.fori_loop(0, N // 16, zinit, 0)

        io = lax.iota(jnp.int32, 16)
        bufs = ((gs0, gd0, sem0), (gs1, gd1, sem1))

        def issue(ci, b):
            gs, gd, sem = bufs[b]
            pltpu.async_copy(gmat.at[srcall.at[pl.ds(ci * CHUNK, CHUNK)]],
                             gs, sem)
            pltpu.async_copy(rmat.at[dstall.at[pl.ds(ci * CHUNK, CHUNK)]],
                             gd, sem)

        def waitb(b):
            gs, gd, sem = bufs[b]
            pltpu.make_async_copy(
                gmat.at[srcall.at[pl.ds(0, CHUNK)]], gs, sem).wait()
            pltpu.make_async_copy(
                rmat.at[dstall.at[pl.ds(0, CHUNK)]], gd, sem).wait()

        def compute(ci, b):
            gs, gd, _ = bufs[b]
            ebl = ci * CHUNK
            for g in range(NGRP):
                rows = g * 16 + io
                dots = jnp.zeros((16,), jnp.float32)
                for l in range(16):
                    er = g * 16 + l
                    parts = [gs[er, pl.ds(r * 16, 16)] *
                             gd[er, pl.ds(r * 16, 16)]
                             for r in range(h // 16)]
                    while len(parts) > 1:
                        parts = [parts[i] + parts[i + 1]
                                 for i in range(0, len(parts) - 1, 2)] + (
                                     [parts[-1]] if len(parts) % 2 else [])
                    dots = jnp.where(io == l, jnp.sum(parts[0]), dots)
                s16 = srcall[pl.ds(ebl + g * 16, 16)]
                d16 = dstall[pl.ds(ebl + g * 16, 16)]
                a = (dots +
                     plsc.load_gather(atab, [s16]) +
                     plsc.load_gather(btab, [d16]))
                logall[pl.ds(ebl + g * 16, 16)] = a
                ks, vs = plsc.sort_key_val(d16, a)
                for st in (1, 2, 4, 8):
                    idxs = jnp.maximum(io - st, 0)
                    vsh = _dg(vs, idxs)
                    ksh = _dg(ks, idxs)
                    ok = (ksh == ks) & (io >= st)
                    vs = jnp.where(ok, jnp.maximum(vs, vsh), vs)
                kn = _dg(ks, jnp.minimum(io + 1, 15))
                is_end = (io == 15) | (kn != ks)
                old = plsc.load_gather(mxtab, [ks], mask=is_end)
                plsc.store_scatter(mxtab, [ks], jnp.maximum(old, vs),
                                   mask=is_end)

        issue(0, 0)
        issue(1, 1)

        def pair(jj, c):
            ci0 = 2 * jj
            waitb(0)
            compute(ci0, 0)
            issue(ci0 + 2, 0)
            ci1 = 2 * jj + 1
            waitb(1)
            compute(ci1, 1)

            @pl.when(ci1 + 2 < NCHUNK)
            def _():
                issue(ci1 + 2, 1)
            return c
        lax.fori_loop(0, NCHUNK // 2, pair, 0)
        waitb(0)
        compute(NCHUNK - 1, 0)

        pltpu.sync_copy(mxtab, mxpart.at[pl.ds(wid * N, N)])
        pltpu.sync_copy(logall, logits.at[pl.ds(base_e, EPW)])

    return pl.kernel(
        body,
        out_type=(jax.ShapeDtypeStruct((E,), jnp.float32),
                  jax.ShapeDtypeStruct((NW * N,), jnp.float32)),
        mesh=_MESH,
        compiler_params=_SC_PARAMS,
        scratch_types=[
            pltpu.VMEM((N,), jnp.float32),
            pltpu.VMEM((N,), jnp.float32),
            pltpu.VMEM((N,), jnp.float32),
            pltpu.VMEM((EPW,), jnp.int32),
            pltpu.VMEM((EPW,), jnp.int32),
            pltpu.VMEM((EPW,), jnp.float32),
            pltpu.VMEM((CHUNK, WR), jnp.float32),
            pltpu.VMEM((CHUNK, WR), jnp.float32),
            pltpu.VMEM((CHUNK, WR), jnp.float32),
            pltpu.VMEM((CHUNK, WR), jnp.float32),
            pltpu.SemaphoreType.DMA,
            pltpu.SemaphoreType.DMA,
        ],
    )


def _make_sc_agg(f):
    def body(esrc, edst, gmat, logits, mxf, accden,
             mxtab, sv0, sv1, dv0, dv1, lv0, lv1, dc0, dc1,
             fs0, fs1, zb, accsp,
             semi0, semi1, semf0, semf1, sems0, sems1):
        cid = lax.axis_index("c")
        sid = lax.axis_index("s")
        base_e = (sid * NC + cid) * EPW
        pltpu.sync_copy(mxf.at[pl.ds(0, N)], mxtab)

        def zrow(i, c):
            for j in range(WR // 16):
                zb[i, pl.ds(j * 16, 16)] = jnp.zeros((16,), jnp.float32)
            return c
        lax.fori_loop(0, ZR, zrow, 0)
        for t in range(SPR // ZR):
            pltpu.sync_copy(zb, accsp.at[pl.ds(sid * SPR + t * ZR, ZR)])
        plsc.subcore_barrier()

        io = lax.iota(jnp.int32, 16)

        ibufs = ((sv0, dv0, lv0, semi0), (sv1, dv1, lv1, semi1))
        fbufs = ((fs0, semf0), (fs1, semf1))
        sbufs = ((fs0, dc0, sems0), (fs1, dc1, sems1))

        def issue_i(ci, b):
            sv, dv, lv, sem = ibufs[b]
            eb = base_e + ci * CHUNK
            pltpu.async_copy(esrc.at[pl.ds(eb, CHUNK)], sv, sem)
            pltpu.async_copy(edst.at[pl.ds(eb, CHUNK)], dv, sem)
            pltpu.async_copy(logits.at[pl.ds(eb, CHUNK)], lv, sem)

        def wait_i(b):
            sv, dv, lv, sem = ibufs[b]
            pltpu.make_async_copy(esrc.at[pl.ds(0, CHUNK)], sv, sem).wait()
            pltpu.make_async_copy(edst.at[pl.ds(0, CHUNK)], dv, sem).wait()
            pltpu.make_async_copy(logits.at[pl.ds(0, CHUNK)], lv, sem).wait()

        def issue_f(b):
            sv = ibufs[b][0]
            fs, sem = fbufs[b]
            pltpu.async_copy(gmat.at[sv], fs, sem)

        def wait_f(b):
            sv = ibufs[b][0]
            fs, sem = fbufs[b]
            pltpu.make_async_copy(gmat.at[sv], fs, sem).wait()

        def issue_s(b):
            fs, dc, sem = sbufs[b]
            pltpu.async_copy(fs, accsp.at[dc], sem, add=True)

        def wait_s(b):
            fs, dc, sem = sbufs[b]
            pltpu.make_async_copy(fs, accsp.at[dc], sem).wait()

        def compute(b):
            fs, _ = fbufs[b]
            dv = ibufs[b][1]
            lv = ibufs[b][2]
            dc = sbufs[b][1]
            ones = jnp.ones((16,), jnp.float32)
            for g in range(NGRP):
                rows = g * 16 + io
                d16 = dv[pl.ds(g * 16, 16)]
                lg = lv[pl.ds(g * 16, 16)]
                dc[pl.ds(g * 16, 16)] = d16
                m = plsc.load_gather(mxtab, [d16])
                ex = jnp.exp(lg - m)
                ll = jnp.zeros((16,), jnp.int32)
                for l in range(16):
                    er = g * 16 + l
                    sp = _dg(ex, ll)
                    for r in range(f // 16):
                        fs[er, pl.ds(r * 16, 16)] = (
                            fs[er, pl.ds(r * 16, 16)] * sp)
                    if l + 1 < 16:
                        ll = ll + 1
                plsc.store_scatter(fs, [rows, jnp.full((16,), f, jnp.int32)],
                                   ex)
                plsc.store_scatter(fs,
                                   [rows, jnp.full((16,), f + 1, jnp.int32)],
                                   ones)
            issue_s(b)

        issue_i(0, 0)
        issue_i(1, 1)
        wait_i(0)
        issue_f(0)
        wait_f(0)
        compute(0)
        issue_i(2, 0)
        wait_i(1)
        issue_f(1)
        wait_f(1)
        compute(1)
        issue_i(3, 1)
        wait_i(0)
        wait_s(0)
        issue_f(0)

        def pair(jj, c):
            wait_f(0)
            compute(0)
            issue_i(2 * jj + 2, 0)
            wait_i(1)
            wait_s(1)
            issue_f(1)
            wait_f(1)
            compute(1)
            issue_i(2 * jj + 3, 1)
            wait_i(0)
            wait_s(0)
            issue_f(0)
            return c
        lax.fori_loop(1, 61, pair, 0)

        wait_f(0)
        compute(0)
        issue_i(124, 0)
        wait_i(1)
        wait_s(1)
        issue_f(1)
        wait_f(1)
        compute(1)
        wait_i(0)
        wait_s(0)
        issue_f(0)
        wait_f(0)
        compute(0)
        wait_s(1)
        wait_s(0)
        plsc.subcore_barrier()
        pltpu.sync_copy(accsp.at[pl.ds(sid * SPR, SPR)],
                        accden.at[cid, pl.ds(sid * SPR, SPR)])

    return pl.kernel(
        body,
        out_type=jax.ShapeDtypeStruct((NC, NP, WR), jnp.float32),
        mesh=_MESH,
        compiler_params=_SC_PARAMS,
        scratch_types=[
            pltpu.VMEM((N,), jnp.float32),
            pltpu.VMEM((CHUNK,), jnp.int32),
            pltpu.VMEM((CHUNK,), jnp.int32),
            pltpu.VMEM((CHUNK,), jnp.int32),
            pltpu.VMEM((CHUNK,), jnp.int32),
            pltpu.VMEM((CHUNK,), jnp.float32),
            pltpu.VMEM((CHUNK,), jnp.float32),
            pltpu.VMEM((CHUNK,), jnp.int32),
            pltpu.VMEM((CHUNK,), jnp.int32),
            pltpu.VMEM((CHUNK, WR), jnp.float32),
            pltpu.VMEM((CHUNK, WR), jnp.float32),
            pltpu.VMEM((ZR, WR), jnp.float32),
            pltpu.VMEM_SHARED((NP, WR), jnp.float32),
            pltpu.SemaphoreType.DMA,
            pltpu.SemaphoreType.DMA,
            pltpu.SemaphoreType.DMA,
            pltpu.SemaphoreType.DMA,
            pltpu.SemaphoreType.DMA,
            pltpu.SemaphoreType.DMA,
        ],
    )


_SCL1 = _make_sc_logits(NHID + NCLASS)
_SCL2 = _make_sc_logits(2 * NCLASS)
_SCA1 = _make_sc_agg(NHID)
_SCA2 = _make_sc_agg(NCLASS)


def _layer_scalars(beta, aw, eps, sa):
    w = jax.nn.softmax(aw, axis=1)
    w0 = w[0, 0]
    w1 = w[0, 1]
    bw = 2.0 / (jnp.exp(-beta[0, 0]) + 1.0)
    c_ff = 2.0 * bw * w0
    c_f2 = bw * w0
    c_l2 = bw * w1
    glg = sa + 2.0 * bw * w1
    c_eps = 1e-9 / (jnp.exp(-eps[0, 0]) + 1.0)
    return c_ff, c_f2, c_l2, glg, c_eps


def _pack8(*vals):
    v = list(vals) + [jnp.float32(0.0)] * (8 - len(vals))
    return jnp.stack([jnp.asarray(x, jnp.float32) for x in v]).reshape(1, 8)


def kernel(x, edge_index, latp, W1, attn_l1, attn_r1, s_attn1, beta1, aw1,
           eps1, W2, attn_l2, attn_r2, s_attn2, beta2, aw2, eps2):
    esrc = edge_index[0].astype(jnp.int32)
    edst = edge_index[1].astype(jnp.int32)
    c_ff1, c_f21, c_l21, glg1, c_eps1 = _layer_scalars(beta1, aw1, eps1,
                                                       s_attn1)
    c_ff2, c_f22, c_l22, glg2, c_eps2 = _layer_scalars(beta2, aw2, eps2,
                                                       s_attn2)
    scal0 = _pack8(c_ff1, c_f21, c_l21)
    scal2 = _pack8(c_ff2, c_f22, c_l22, c_eps1)
    scal4 = _pack8(c_eps2)

    g1, r1, f1 = _P0(x, latp, W1.T, glg1, scal0)
    ns1 = _NST1(f1, latp, attn_l1.reshape(1, NHID),
                attn_r1.reshape(1, NHID), scal0)
    logits1, mxp1 = _SCL1(esrc, edst, g1, r1, ns1.reshape(-1))
    mx1 = _PMX(mxp1.reshape(NW, N))
    ad1 = _SCA1(esrc, edst, g1, logits1, mx1.reshape(-1))
    g2, r2, f2 = _P2(ad1, f1, latp, W2.T, glg2, scal2)
    ns2 = _NST2(f2, latp, attn_l2.reshape(1, NCLASS),
                attn_r2.reshape(1, NCLASS), scal2)
    logits2, mxp2 = _SCL2(esrc, edst, g2, r2, ns2.reshape(-1))
    mx2 = _PMX(mxp2.reshape(NW, N))
    ad2 = _SCA2(esrc, edst, g2, logits2, mx2.reshape(-1))
    out = _P4(ad2, f2, scal4)
    return out
